# baseline (speedup 1.0000x reference)
import functools
import math

import jax
import jax.numpy as jnp
from jax import lax
from jax.experimental import pallas as pl
from jax.experimental.pallas import tpu as pltpu
from jax.experimental.pallas import tpu_sc as plsc

VOCAB = 1000000
D_MODEL = 64
SCALE = math.sqrt(D_MODEL)

NUM_CORES = 2
NUM_SUBCORES = 16
NW = NUM_CORES * NUM_SUBCORES

B_TOTAL = 4096 * 200
BPW = B_TOTAL // NW
CHUNK = 128
NCHUNK = BPW // CHUNK
NBUF = 4


def _sc_embedding_lookup(idx_flat, emb_weight):
    mesh = plsc.VectorSubcoreMesh(core_axis_name="c", subcore_axis_name="s")

    scratch = (
        [pltpu.VMEM((BPW,), jnp.int32)]
        + [pltpu.VMEM((CHUNK, D_MODEL), jnp.float32) for _ in range(2 * NBUF)]
        + [pltpu.SemaphoreType.DMA for _ in range(2 * NBUF)]
    )

    @functools.partial(
        pl.kernel,
        mesh=mesh,
        out_type=jax.ShapeDtypeStruct((B_TOTAL, D_MODEL), jnp.float32),
        scratch_types=scratch,
        compiler_params=pltpu.CompilerParams(use_tc_tiling_on_sc=False),
    )
    def k(idx_hbm, table_hbm, out_hbm, idx_v, *rest):
        gbuf = rest[0:NBUF]
        sbuf = rest[NBUF:2 * NBUF]
        gsem = rest[2 * NBUF:3 * NBUF]
        ssem = rest[3 * NBUF:4 * NBUF]

        wid = lax.axis_index("s") * NUM_CORES + lax.axis_index("c")
        base = wid * BPW

        pltpu.sync_copy(idx_hbm.at[pl.ds(base, BPW)], idx_v)

        def gather_start(g, b):
            pltpu.make_async_copy(
                table_hbm.at[idx_v.at[pl.ds(g * CHUNK, CHUNK)]], gbuf[b], gsem[b]
            ).start()

        def gather_wait(b):
            pltpu.make_async_copy(
                table_hbm.at[idx_v.at[pl.ds(0, CHUNK)]], gbuf[b], gsem[b]
            ).wait()

        def scatter_start(g, b):
            pltpu.make_async_copy(
                sbuf[b], out_hbm.at[pl.ds(base + g * CHUNK, CHUNK)], ssem[b]
            ).start()

        def scatter_wait(b):
            pltpu.make_async_copy(
                sbuf[b], out_hbm.at[pl.ds(base, CHUNK)], ssem[b]
            ).wait()

        for b in range(NBUF):
            gather_start(b, b)

        def outer(t, carry):
            g0 = t * NBUF
            for b in range(NBUF):
                g = g0 + b
                gather_wait(b)

                @pl.when(g >= NBUF)
                def _():
                    scatter_wait(b)

                def scale_row(r, c):
                    for j in range(D_MODEL // 16):
                        sbuf[b][r, pl.ds(j * 16, 16)] = (
                            gbuf[b][r, pl.ds(j * 16, 16)] * SCALE
                        )
                    return c

                lax.fori_loop(0, CHUNK, scale_row, 0, unroll=2)

                scatter_start(g, b)

                @pl.when(g + NBUF < NCHUNK)
                def _():
                    gather_start(g + NBUF, b)
            return carry

        lax.fori_loop(0, NCHUNK // NBUF, outer, 0)

        for b in range(NBUF):
            scatter_wait(b)

    return k(idx_flat, emb_weight)


def kernel(x, emb_weight):
    idx_flat = x.reshape(-1).astype(jnp.int32)
    out = _sc_embedding_lookup(idx_flat, emb_weight)
    return out.reshape(x.shape + (D_MODEL,))


# speedup vs baseline: 1.0009x; 1.0009x over previous
import functools
import math

import jax
import jax.numpy as jnp
from jax import lax
from jax.experimental import pallas as pl
from jax.experimental.pallas import tpu as pltpu
from jax.experimental.pallas import tpu_sc as plsc

VOCAB = 1000000
D_MODEL = 64
SCALE = math.sqrt(D_MODEL)

NUM_CORES = 2
NUM_SUBCORES = 16
NW = NUM_CORES * NUM_SUBCORES

ROWS = 4096
SEQ = 200
RPW = ROWS // NW
NBUF = 4
SPLITS = ((0, 128), (128, 72))


def _sc_embedding_lookup(x, emb_weight):
    mesh = plsc.VectorSubcoreMesh(core_axis_name="c", subcore_axis_name="s")

    scratch = (
        [pltpu.VMEM((RPW, SEQ), jnp.int32)]
        + [pltpu.VMEM((SEQ, D_MODEL), jnp.float32) for _ in range(2 * NBUF)]
        + [pltpu.SemaphoreType.DMA for _ in range(2 * NBUF)]
    )

    @functools.partial(
        pl.kernel,
        mesh=mesh,
        out_type=jax.ShapeDtypeStruct((ROWS, SEQ, D_MODEL), jnp.float32),
        scratch_types=scratch,
        compiler_params=pltpu.CompilerParams(use_tc_tiling_on_sc=False),
    )
    def k(x_hbm, table_hbm, out_hbm, idx_v, *rest):
        gbuf = rest[0:NBUF]
        sbuf = rest[NBUF:2 * NBUF]
        gsem = rest[2 * NBUF:3 * NBUF]
        ssem = rest[3 * NBUF:4 * NBUF]

        wid = lax.axis_index("s") * NUM_CORES + lax.axis_index("c")
        row0 = wid * RPW

        pltpu.sync_copy(x_hbm.at[pl.ds(row0, RPW)], idx_v)

        def gather_start(r, b):
            for off, n in SPLITS:
                pltpu.make_async_copy(
                    table_hbm.at[idx_v.at[r, pl.ds(off, n)]],
                    gbuf[b].at[pl.ds(off, n)],
                    gsem[b],
                ).start()

        def gather_wait(b):
            for off, n in SPLITS:
                pltpu.make_async_copy(
                    table_hbm.at[idx_v.at[0, pl.ds(off, n)]],
                    gbuf[b].at[pl.ds(off, n)],
                    gsem[b],
                ).wait()

        def scatter_start(r, b):
            pltpu.make_async_copy(
                sbuf[b], out_hbm.at[row0 + r], ssem[b]
            ).start()

        def scatter_wait(b):
            pltpu.make_async_copy(
                sbuf[b], out_hbm.at[row0], ssem[b]
            ).wait()

        for b in range(NBUF):
            gather_start(b, b)

        def outer(t0, carry):
            for b in range(NBUF):
                r = t0 * NBUF + b
                gather_wait(b)

                @pl.when(r >= NBUF)
                def _():
                    scatter_wait(b)

                def scale_row(i, c):
                    for j in range(D_MODEL // 16):
                        sbuf[b][i, pl.ds(j * 16, 16)] = (
                            gbuf[b][i, pl.ds(j * 16, 16)] * SCALE
                        )
                    return c

                lax.fori_loop(0, SEQ, scale_row, 0, unroll=2)

                scatter_start(r, b)

                @pl.when(r + NBUF < RPW)
                def _():
                    gather_start(r + NBUF, b)
            return carry

        lax.fori_loop(0, RPW // NBUF, outer, 0)

        for b in range(NBUF):
            scatter_wait(b)

    return k(x, emb_weight)


def kernel(x, emb_weight):
    return _sc_embedding_lookup(x.astype(jnp.int32), emb_weight)
